# baseline (speedup 1.0000x reference)
import functools

import jax
import jax.numpy as jnp
from jax import lax
from jax.experimental import pallas as pl
from jax.experimental.pallas import tpu as pltpu
from jax.experimental.pallas import tpu_sc as plsc

N_NODES = 10000
N_PAD = 10240
D_FEAT = 256
QUADS = D_FEAT // 4
ROWW = QUADS + 8
L = 16
NC = 2
NS = 16
NW = NC * NS
C = 128
F8 = jnp.float8_e4m3fn


def _normalize_tc(z):
    def body(z_ref, o_ref):
        x = z_ref[...]
        xn = x * lax.rsqrt(jnp.sum(x * x, axis=1, keepdims=True))
        b = lax.bitcast_convert_type(xn.astype(F8), jnp.uint8
                                     ).astype(jnp.int32)
        Q = QUADS
        w = (b[:, 0:Q] | (b[:, Q:2 * Q] << 8) | (b[:, 2 * Q:3 * Q] << 16)
             | (b[:, 3 * Q:4 * Q] << 24))
        o_ref[pl.ds(0, N_NODES), pl.ds(0, QUADS)] = w
        o_ref[pl.ds(0, N_NODES), pl.ds(QUADS, 8)] = w[:, 0:8]

    return pl.pallas_call(
        body,
        out_shape=jax.ShapeDtypeStruct((N_PAD, ROWW), jnp.int32),
    )(z)


def _vsqrt(s):
    i = plsc.bitcast(s, jnp.int32)
    y = plsc.bitcast(jnp.int32(0x5F3759DF) - lax.shift_right_arithmetic(i, 1),
                     jnp.float32)
    for _ in range(3):
        y = y * (1.5 - 0.5 * s * y * y)
    return s * y


def _make_sc_kernel(n_edges):
    assert n_edges % (2 * C) == 0
    pairs_total = n_edges // (2 * C)
    ppw = -(-pairs_total // NW)
    last_pairs = pairs_total - (NW - 1) * ppw
    assert last_pairs > 0
    epw_full = ppw * 2 * C
    epw_last = last_pairs * 2 * C

    mesh = plsc.VectorSubcoreMesh(
        core_axis_name="c", subcore_axis_name="s",
        num_cores=NC, num_subcores=NS)

    @functools.partial(
        pl.kernel,
        mesh=mesh,
        compiler_params=pltpu.CompilerParams(use_tc_tiling_on_sc=False,
                                             needs_layout_passes=False),
        out_type=jax.ShapeDtypeStruct((n_edges,), jnp.float32),
        scratch_types=[
            pltpu.VMEM_SHARED((N_PAD, ROWW), jnp.int32),
            pltpu.VMEM((epw_full,), jnp.int32),
            pltpu.VMEM((epw_full,), jnp.int32),
            pltpu.VMEM((C, ROWW), jnp.int32),
            pltpu.VMEM((C, ROWW), jnp.int32),
            pltpu.VMEM((C, ROWW), jnp.int32),
            pltpu.VMEM((C, ROWW), jnp.int32),
            pltpu.VMEM((C,), jnp.float32),
            pltpu.VMEM((C,), jnp.float32),
            pltpu.SemaphoreType.DMA, pltpu.SemaphoreType.DMA,
            pltpu.SemaphoreType.DMA, pltpu.SemaphoreType.DMA,
            pltpu.SemaphoreType.DMA, pltpu.SemaphoreType.DMA,
        ],
    )
    def sc_kernel(zpk_hbm, ei_hbm, out_hbm,
                  ztab, src_all, dst_all, a0, a1, b0, b1, o0, o1,
                  sa0, sa1, sb0, sb1, so0, so1):
        sid = lax.axis_index("s")
        wid = sid * NC + lax.axis_index("c")
        wbase = wid * epw_full
        mypairs = jnp.minimum(ppw, pairs_total - wid * ppw)
        nchunk = 2 * mypairs
        avs = (a0, a1)
        bvs = (b0, b1)
        outs = (o0, o1)
        sas = (sa0, sa1)
        sbs = (sb0, sb1)
        sos = (so0, so1)

        rpt = N_PAD // NS
        pltpu.sync_copy(zpk_hbm.at[pl.ds(sid * rpt, rpt)],
                        ztab.at[pl.ds(sid * rpt, rpt)])

        @pl.when(wid < NW - 1)
        def _():
            pltpu.sync_copy(ei_hbm.at[0, pl.ds(wbase, epw_full)], src_all)
            pltpu.sync_copy(ei_hbm.at[1, pl.ds(wbase, epw_full)], dst_all)

        @pl.when(wid == NW - 1)
        def _():
            pltpu.sync_copy(ei_hbm.at[0, pl.ds(wbase, epw_last)],
                            src_all.at[pl.ds(0, epw_last)])
            pltpu.sync_copy(ei_hbm.at[1, pl.ds(wbase, epw_last)],
                            dst_all.at[pl.ds(0, epw_last)])

        plsc.subcore_barrier()

        def start_gather(c, p):
            off = c * C
            pltpu.async_copy(
                ztab.at[src_all.at[pl.ds(off, C)]], avs[p], sas[p])
            pltpu.async_copy(
                ztab.at[dst_all.at[pl.ds(off, C)]], bvs[p], sbs[p])

        def wait_gather(c, p):
            off = c * C
            pltpu.make_async_copy(
                ztab.at[src_all.at[pl.ds(off, C)]], avs[p], sas[p]).wait()
            pltpu.make_async_copy(
                ztab.at[dst_all.at[pl.ds(off, C)]], bvs[p], sbs[p]).wait()

        lane = lax.iota(jnp.int32, L)

        def compute(c, p):
            a_ref, b_ref = avs[p], bvs[p]
            out_v = outs[p]

            zero16 = jnp.zeros((L,), jnp.int32)

            def gbody(g, _):
                eids = g * L + lane
                ebase = eids * ROWW

                def obody(o, carry):
                    lo, hi = carry
                    acc0 = None
                    acc1 = None
                    f0 = ebase + ((o * 8 + lane) & (QUADS - 1))
                    for k in range(8):
                        fv = f0 + k
                        ai = plsc.load_gather(a_ref, [zero16, fv])
                        bi = plsc.load_gather(b_ref, [zero16, fv])
                        a0, a1 = plsc.unpack(
                            plsc.bitcast(ai, F8),
                            format=plsc.PackFormat.INTERLEAVED,
                            preferred_element_type=jnp.bfloat16)
                        b0, b1 = plsc.unpack(
                            plsc.bitcast(bi, F8),
                            format=plsc.PackFormat.INTERLEAVED,
                            preferred_element_type=jnp.bfloat16)
                        p0 = a0 * b0
                        p1 = a1 * b1
                        acc0 = p0 if acc0 is None else acc0 + p0
                        acc1 = p1 if acc1 is None else acc1 + p1
                    dlo, dhi = plsc.unpack(
                        acc0 + acc1, format=plsc.PackFormat.INTERLEAVED)
                    return lo + dlo, hi + dhi

                z16 = jnp.zeros((L,), jnp.float32)
                lo, hi = lax.fori_loop(0, QUADS // 8, obody, (z16, z16),
                                       unroll=2)
                s = jnp.maximum(2.0 - 2.0 * (lo + hi), 0.0)
                dist = _vsqrt(s)
                out_v[pl.ds(g * L, L)] = 1.0 / (1.0 + jnp.exp(dist - 1.0))
                return 0

            lax.fori_loop(0, C // L, gbody, 0, unroll=False)
            pltpu.async_copy(out_v, out_hbm.at[pl.ds(wbase + c * C, C)],
                             sos[p])

        def wait_out(c, p):
            pltpu.make_async_copy(
                outs[p], out_hbm.at[pl.ds(wbase + c * C, C)], sos[p]).wait()

        start_gather(0, 0)

        def pair_body(t, _):
            for parity in (0, 1):
                c = 2 * t + parity
                nxt = 1 - parity

                wait_gather(c, parity)

                @pl.when(c + 1 < nchunk)
                def _():
                    start_gather(c + 1, nxt)

                @pl.when(c >= 2)
                def _():
                    wait_out(c - 2, parity)

                compute(c, parity)
            return 0

        lax.fori_loop(0, mypairs, pair_body, 0, unroll=False)
        wait_out(nchunk - 2, 0)
        wait_out(nchunk - 1, 1)

    return sc_kernel


_SC_KERNEL_CACHE = {}


def kernel(z, edge_index):
    zpk = _normalize_tc(z)
    n_edges = edge_index.shape[1]
    if n_edges not in _SC_KERNEL_CACHE:
        _SC_KERNEL_CACHE[n_edges] = _make_sc_kernel(n_edges)
    return _SC_KERNEL_CACHE[n_edges](zpk, edge_index)


# speedup vs baseline: 1.0319x; 1.0319x over previous
import functools

import jax
import jax.numpy as jnp
from jax import lax
from jax.experimental import pallas as pl
from jax.experimental.pallas import tpu as pltpu
from jax.experimental.pallas import tpu_sc as plsc

N_NODES = 10000
N_PAD = 10240
D_FEAT = 256
QUADS = D_FEAT // 4
L = 16
NC = 2
NS = 16
NW = NC * NS
C = 128
F8 = jnp.float8_e4m3fn


def _normalize_tc(z):
    def body(z_ref, o_ref):
        x = z_ref[...]
        xn = x * lax.rsqrt(jnp.sum(x * x, axis=1, keepdims=True))
        b = lax.bitcast_convert_type(xn.astype(F8), jnp.uint8
                                     ).astype(jnp.int32)
        Q = QUADS
        w = (b[:, 0:Q] | (b[:, Q:2 * Q] << 8) | (b[:, 2 * Q:3 * Q] << 16)
             | (b[:, 3 * Q:4 * Q] << 24))
        o_ref[pl.ds(0, N_NODES), :] = w

    return pl.pallas_call(
        body,
        out_shape=jax.ShapeDtypeStruct((N_PAD, QUADS), jnp.int32),
    )(z)


def _vsqrt(s):
    i = plsc.bitcast(s, jnp.int32)
    y = plsc.bitcast(jnp.int32(0x5F3759DF) - lax.shift_right_arithmetic(i, 1),
                     jnp.float32)
    for _ in range(3):
        y = y * (1.5 - 0.5 * s * y * y)
    return s * y


def _make_sc_kernel(n_edges):
    assert n_edges % (2 * C) == 0
    pairs_total = n_edges // (2 * C)
    ppw = -(-pairs_total // NW)
    last_pairs = pairs_total - (NW - 1) * ppw
    assert last_pairs > 0
    epw_full = ppw * 2 * C
    epw_last = last_pairs * 2 * C

    mesh = plsc.VectorSubcoreMesh(
        core_axis_name="c", subcore_axis_name="s",
        num_cores=NC, num_subcores=NS)

    @functools.partial(
        pl.kernel,
        mesh=mesh,
        compiler_params=pltpu.CompilerParams(use_tc_tiling_on_sc=False,
                                             needs_layout_passes=False),
        out_type=jax.ShapeDtypeStruct((n_edges,), jnp.float32),
        scratch_types=[
            pltpu.VMEM_SHARED((N_PAD, QUADS), jnp.int32),
            pltpu.VMEM((epw_full,), jnp.int32),
            pltpu.VMEM((epw_full,), jnp.int32),
            pltpu.VMEM((C, QUADS), jnp.int32),
            pltpu.VMEM((C, QUADS), jnp.int32),
            pltpu.VMEM((C, QUADS), jnp.int32),
            pltpu.VMEM((C, QUADS), jnp.int32),
            pltpu.VMEM((C,), jnp.float32),
            pltpu.VMEM((C,), jnp.float32),
            pltpu.SemaphoreType.DMA, pltpu.SemaphoreType.DMA,
            pltpu.SemaphoreType.DMA, pltpu.SemaphoreType.DMA,
            pltpu.SemaphoreType.DMA, pltpu.SemaphoreType.DMA,
            pltpu.SemaphoreType.DMA,
        ],
    )
    def sc_kernel(zpk_hbm, ei_hbm, out_hbm,
                  ztab, src_all, dst_all, a0, a1, b0, b1, o0, o1,
                  sa0, sa1, sb0, sb1, so0, so1, s_st):
        sid = lax.axis_index("s")
        wid = sid * NC + lax.axis_index("c")
        wbase = wid * epw_full
        mypairs = jnp.minimum(ppw, pairs_total - wid * ppw)
        nchunk = 2 * mypairs
        avs = (a0, a1)
        bvs = (b0, b1)
        outs = (o0, o1)
        sas = (sa0, sa1)
        sbs = (sb0, sb1)
        sos = (so0, so1)

        rpt = N_PAD // NS
        pltpu.async_copy(zpk_hbm.at[pl.ds(sid * rpt, rpt)],
                         ztab.at[pl.ds(sid * rpt, rpt)], s_st)

        @pl.when(wid < NW - 1)
        def _():
            pltpu.sync_copy(ei_hbm.at[0, pl.ds(wbase, epw_full)], src_all)
            pltpu.sync_copy(ei_hbm.at[1, pl.ds(wbase, epw_full)], dst_all)

        @pl.when(wid == NW - 1)
        def _():
            pltpu.sync_copy(ei_hbm.at[0, pl.ds(wbase, epw_last)],
                            src_all.at[pl.ds(0, epw_last)])
            pltpu.sync_copy(ei_hbm.at[1, pl.ds(wbase, epw_last)],
                            dst_all.at[pl.ds(0, epw_last)])

        K = 8

        def start_gather(c, p):
            off = c * C

            @pl.when(c < K)
            def _():
                pltpu.async_copy(
                    zpk_hbm.at[src_all.at[pl.ds(off, C)]], avs[p], sas[p])
                pltpu.async_copy(
                    zpk_hbm.at[dst_all.at[pl.ds(off, C)]], bvs[p], sbs[p])

            @pl.when(c >= K)
            def _():
                pltpu.async_copy(
                    ztab.at[src_all.at[pl.ds(off, C)]], avs[p], sas[p])
                pltpu.async_copy(
                    ztab.at[dst_all.at[pl.ds(off, C)]], bvs[p], sbs[p])

        def wait_gather(c, p):
            off = c * C

            @pl.when(c < K)
            def _():
                pltpu.make_async_copy(
                    zpk_hbm.at[src_all.at[pl.ds(off, C)]],
                    avs[p], sas[p]).wait()
                pltpu.make_async_copy(
                    zpk_hbm.at[dst_all.at[pl.ds(off, C)]],
                    bvs[p], sbs[p]).wait()

            @pl.when(c >= K)
            def _():
                pltpu.make_async_copy(
                    ztab.at[src_all.at[pl.ds(off, C)]],
                    avs[p], sas[p]).wait()
                pltpu.make_async_copy(
                    ztab.at[dst_all.at[pl.ds(off, C)]],
                    bvs[p], sbs[p]).wait()

        lane = lax.iota(jnp.int32, L)

        def compute(c, p):
            a_ref, b_ref = avs[p], bvs[p]
            out_v = outs[p]

            def gbody(g, _):
                eids = g * L + lane

                def obody(o, carry):
                    lo, hi = carry
                    acc0 = None
                    acc1 = None
                    for k in range(8):
                        qv = (o * 8 + k + lane) & (QUADS - 1)
                        ai = plsc.load_gather(a_ref, [eids, qv])
                        bi = plsc.load_gather(b_ref, [eids, qv])
                        a0, a1 = plsc.unpack(
                            plsc.bitcast(ai, F8),
                            format=plsc.PackFormat.INTERLEAVED,
                            preferred_element_type=jnp.bfloat16)
                        b0, b1 = plsc.unpack(
                            plsc.bitcast(bi, F8),
                            format=plsc.PackFormat.INTERLEAVED,
                            preferred_element_type=jnp.bfloat16)
                        p0 = a0 * b0
                        p1 = a1 * b1
                        acc0 = p0 if acc0 is None else acc0 + p0
                        acc1 = p1 if acc1 is None else acc1 + p1
                    dlo, dhi = plsc.unpack(
                        acc0 + acc1, format=plsc.PackFormat.INTERLEAVED)
                    return lo + dlo, hi + dhi

                z16 = jnp.zeros((L,), jnp.float32)
                lo, hi = lax.fori_loop(0, QUADS // 8, obody, (z16, z16),
                                       unroll=2)
                s = jnp.maximum(2.0 - 2.0 * (lo + hi), 0.0)
                dist = _vsqrt(s)
                out_v[pl.ds(g * L, L)] = 1.0 / (1.0 + jnp.exp(dist - 1.0))
                return 0

            lax.fori_loop(0, C // L, gbody, 0, unroll=False)
            pltpu.async_copy(out_v, out_hbm.at[pl.ds(wbase + c * C, C)],
                             sos[p])

        def wait_out(c, p):
            pltpu.make_async_copy(
                outs[p], out_hbm.at[pl.ds(wbase + c * C, C)], sos[p]).wait()

        start_gather(0, 0)

        def pair_body(t, _):
            for parity in (0, 1):
                c = 2 * t + parity
                nxt = 1 - parity

                wait_gather(c, parity)

                @pl.when(c + 1 == K)
                def _():
                    pltpu.make_async_copy(
                        zpk_hbm.at[pl.ds(sid * rpt, rpt)],
                        ztab.at[pl.ds(sid * rpt, rpt)], s_st).wait()
                    plsc.subcore_barrier()

                @pl.when(c + 1 < nchunk)
                def _():
                    start_gather(c + 1, nxt)

                @pl.when(c >= 2)
                def _():
                    wait_out(c - 2, parity)

                compute(c, parity)
            return 0

        lax.fori_loop(0, mypairs, pair_body, 0, unroll=False)
        wait_out(nchunk - 2, 0)
        wait_out(nchunk - 1, 1)

    return sc_kernel


_SC_KERNEL_CACHE = {}


def kernel(z, edge_index):
    zpk = _normalize_tc(z)
    n_edges = edge_index.shape[1]
    if n_edges not in _SC_KERNEL_CACHE:
        _SC_KERNEL_CACHE[n_edges] = _make_sc_kernel(n_edges)
    return _SC_KERNEL_CACHE[n_edges](zpk, edge_index)
